# baseline (speedup 1.0000x reference)
import functools

import jax
import jax.numpy as jnp
from jax import lax
from jax.experimental import pallas as pl
from jax.experimental.pallas import tpu as pltpu
from jax.experimental.pallas import tpu_sc as plsc

_N_NODE = 1000000
_IDX_CHUNK = 128


@functools.cache
def _make_sc_gather(NT2, RT2, B):
    info = plsc.get_sparse_core_info()
    NC, NS = info.num_cores, info.num_subcores
    NW = NC * NS
    b_per_w = B // NW
    n_chunks = b_per_w // _IDX_CHUNK
    mesh = plsc.VectorSubcoreMesh(core_axis_name="c", subcore_axis_name="s")

    @functools.partial(
        pl.kernel,
        out_type=(
            jax.ShapeDtypeStruct((B, 128), jnp.float32),
            jax.ShapeDtypeStruct((B, 128), jnp.float32),
        ),
        mesh=mesh,
        scratch_types=[
            pltpu.VMEM((b_per_w,), jnp.int32),
            pltpu.VMEM((b_per_w,), jnp.int32),
            pltpu.VMEM((b_per_w, 128), jnp.float32),
            pltpu.SemaphoreType.DMA,
        ],
    )
    def sc_gather(node_hbm, rel_hbm, eidx_hbm, base_hbm, ht_out, rel_out,
                  eidx_v, base_v, rows_v, sem):
        wid = lax.axis_index("s") * NC + lax.axis_index("c")
        off = wid * b_per_w
        pltpu.sync_copy(eidx_hbm.at[pl.ds(off, b_per_w)], eidx_v)
        pltpu.sync_copy(base_hbm.at[pl.ds(off, b_per_w)], base_v)
        for j in range(b_per_w // 16):
            sl = pl.ds(j * 16, 16)
            eidx_v[sl] = lax.shift_right_arithmetic(eidx_v[sl] - _N_NODE, 1)
            base_v[sl] = lax.shift_right_arithmetic(base_v[sl], 1)
        copies = []
        for i in range(n_chunks):
            sl = pl.ds(i * _IDX_CHUNK, _IDX_CHUNK)
            copies.append(
                pltpu.async_copy(node_hbm.at[eidx_v.at[sl]], rows_v.at[sl], sem))
        for cp in copies:
            cp.wait()
        pltpu.sync_copy(rows_v, ht_out.at[pl.ds(off, b_per_w)])
        copies = []
        for i in range(n_chunks):
            sl = pl.ds(i * _IDX_CHUNK, _IDX_CHUNK)
            copies.append(
                pltpu.async_copy(rel_hbm.at[base_v.at[sl]], rows_v.at[sl], sem))
        for cp in copies:
            cp.wait()
        pltpu.sync_copy(rows_v, rel_out.at[pl.ds(off, b_per_w)])

    return sc_gather


_TC_BLK = 2048


def _tc_score_body(ht_ref, rel_ref, eidx_ref, base_ref, gt_ref, out_ref, *,
                   inv_b):
    @pl.when(pl.program_id(0) == 0)
    def _():
        out_ref[...] = jnp.zeros_like(out_ref)

    a128 = ht_ref[...]
    b128 = rel_ref[...]
    pa = (eidx_ref[...] - _N_NODE) & 1
    pb = base_ref[...] & 1
    a = jnp.where(pa == 1, a128[:, 64:], a128[:, :64])
    b = jnp.where(pb == 1, b128[:, 64:], b128[:, :64])
    aa = jnp.sum(a * a, axis=1, keepdims=True)
    bb = jnp.sum(b * b, axis=1, keepdims=True)
    ab = jnp.sum(a * b, axis=1, keepdims=True)
    eps = jnp.float32(1e-12)
    denom = jnp.maximum(jnp.sqrt(aa), eps) * jnp.maximum(jnp.sqrt(bb), eps)
    x = -(ab / denom) * gt_ref[...]
    sp = jnp.maximum(x, 0.0) + jnp.log1p(jnp.exp(-jnp.abs(x)))
    out_ref[...] += (jnp.sum(sp) * jnp.float32(inv_b)).reshape(1, 1)


def kernel(node_table, rel_table, base_edge_index, base, ground_truth):
    B = base.shape[0]
    NT, D = node_table.shape
    NR = rel_table.shape[0]
    nt2 = node_table.reshape(NT // 2, 2 * D)
    rt2 = rel_table.reshape(NR // 2, 2 * D)
    eidx = base_edge_index.reshape(B)
    ht128, rel128 = _make_sc_gather(NT // 2, NR // 2, B)(nt2, rt2, eidx, base)
    n_blk = B // _TC_BLK
    loss = pl.pallas_call(
        functools.partial(_tc_score_body, inv_b=1.0 / B),
        grid=(n_blk,),
        in_specs=[
            pl.BlockSpec((_TC_BLK, 128), lambda i: (i, 0)),
            pl.BlockSpec((_TC_BLK, 128), lambda i: (i, 0)),
            pl.BlockSpec((_TC_BLK, 1), lambda i: (i, 0)),
            pl.BlockSpec((_TC_BLK, 1), lambda i: (i, 0)),
            pl.BlockSpec((_TC_BLK, 1), lambda i: (i, 0)),
        ],
        out_specs=pl.BlockSpec((1, 1), lambda i: (0, 0)),
        out_shape=jax.ShapeDtypeStruct((1, 1), jnp.float32),
    )(ht128, rel128, base_edge_index, base.reshape(B, 1), ground_truth)
    return loss[0, 0]
